# baseline (speedup 1.0000x reference)
import functools

import jax
import jax.numpy as jnp
from jax import lax
from jax.experimental import pallas as pl
from jax.experimental.pallas import tpu as pltpu
from jax.experimental.pallas import tpu_sc as plsc

N_NODES = 10000
D = 128
N_EDGES = 320000
E_TOT = N_EDGES
NC = 2
NS = 16
NW = NC * NS
L = 16
CHUNK = 96
CPT = -(-E_TOT // (NW * CHUNK))
WIN = CPT + (-CPT) % 8
GROUP = 8
NG = WIN // GROUP
EPT = CPT * CHUNK
EPAD = NW * EPT
DENP = 10240
DEN_PT = DENP // NS
ROW_PT = 624
ROW_LAST = N_NODES - 15 * ROW_PT


def _prep_body(x_ref, w_ref, a2_ref, tx_ref, s12_ref):
    tx = lax.dot_general(x_ref[...], w_ref[...], (((1,), (1,)), ((), ())),
                         preferred_element_type=jnp.float32)
    tx_ref[...] = tx
    s12_ref[...] = lax.dot_general(a2_ref[...], tx, (((1,), (1,)), ((), ())),
                                   preferred_element_type=jnp.float32)


_prep = pl.pallas_call(
    _prep_body,
    out_shape=(jax.ShapeDtypeStruct((N_NODES, D), jnp.float32),
               jax.ShapeDtypeStruct((2, N_NODES), jnp.float32)),
)


_sc_mesh = plsc.VectorSubcoreMesh(core_axis_name="c", subcore_axis_name="s")


@functools.partial(
    pl.kernel,
    mesh=_sc_mesh,
    compiler_params=pltpu.CompilerParams(use_tc_tiling_on_sc=False,
                                         needs_layout_passes=False),
    out_type=(jax.ShapeDtypeStruct((NC, N_NODES, D), jnp.float32),
              jax.ShapeDtypeStruct((NC, 1, DENP), jnp.float32)),
    scratch_types=[
        pltpu.VMEM((2, GROUP, CHUNK), jnp.int32),
        pltpu.VMEM((2, GROUP, CHUNK), jnp.int32),
        pltpu.VMEM((2, CHUNK), jnp.float32),
        pltpu.VMEM((N_NODES,), jnp.float32),
        pltpu.VMEM((N_NODES,), jnp.float32),
        pltpu.VMEM((2, CHUNK, D), jnp.float32),
        pltpu.VMEM((DEN_PT,), jnp.float32),
        pltpu.VMEM_SHARED((N_NODES, D), jnp.float32),
        pltpu.VMEM_SHARED((DENP,), jnp.float32),
        pltpu.SemaphoreType.DMA,
        pltpu.SemaphoreType.DMA,
        pltpu.SemaphoreType.DMA,
        pltpu.SemaphoreType.DMA,
    ],
)
def _sc_edges(tx_hbm, s1_hbm, s2_hbm, src_hbm, dst_hbm, outp_hbm, denp_hbm,
              src_g, dst_g, num_v, s1_v, s2_v, rows_v, zbuf, out_sh, den_sh,
              gsem, isem, dsem, ssem):
    cid = lax.axis_index("c")
    sid = lax.axis_index("s")
    w = sid * NC + cid
    zv = jnp.zeros((L,), jnp.float32)

    def _zero_rows(r, carry):
        for j in range(D // L):
            rows_v[0, r, pl.ds(j * L, L)] = zv
        return carry

    lax.fori_loop(0, CHUNK, _zero_rows, 0)

    def _zero_zbuf(i, carry):
        zbuf[pl.ds(i * L, L)] = zv
        return carry

    lax.fori_loop(0, DEN_PT // L, _zero_zbuf, 0)

    base = sid * ROW_PT

    @pl.when(sid < NS - 1)
    def _():
        off = 0
        for nrows in (CHUNK, CHUNK, CHUNK, CHUNK, CHUNK, CHUNK,
                      ROW_PT - 6 * CHUNK):
            pltpu.sync_copy(rows_v.at[0, pl.ds(0, nrows)],
                            out_sh.at[pl.ds(base + off, nrows)])
            off += nrows

    @pl.when(sid == NS - 1)
    def _():
        off = 0
        for nrows in (CHUNK, CHUNK, CHUNK, CHUNK, CHUNK, CHUNK,
                      ROW_LAST - 6 * CHUNK):
            pltpu.sync_copy(rows_v.at[0, pl.ds(0, nrows)],
                            out_sh.at[pl.ds(base + off, nrows)])
            off += nrows

    pltpu.sync_copy(zbuf, den_sh.at[pl.ds(sid * DEN_PT, DEN_PT)])
    plsc.subcore_barrier()

    rowbase = w * WIN
    vc = jnp.maximum(jnp.minimum(E_TOT - w * EPT, EPT), 0)

    pltpu.sync_copy(src_hbm.at[pl.ds(rowbase, GROUP)], src_g.at[0])
    pltpu.sync_copy(dst_hbm.at[pl.ds(rowbase, GROUP)], dst_g.at[0])
    pltpu.async_copy(tx_hbm.at[src_g.at[0, 0]], rows_v.at[0], gsem)
    pltpu.sync_copy(s1_hbm, s1_v)
    pltpu.sync_copy(s2_hbm, s2_v)
    pltpu.async_copy(src_hbm.at[pl.ds(rowbase + GROUP, GROUP)], src_g.at[1],
                     isem)
    pltpu.async_copy(dst_hbm.at[pl.ds(rowbase + GROUP, GROUP)], dst_g.at[1],
                     isem)

    def _group(g, carry):
        bg = g & 1
        for k in range(GROUP):
            c = g * GROUP + k
            cb = k & 1
            if k >= 2:
                pltpu.make_async_copy(num_v.at[cb],
                                      den_sh.at[dst_g.at[bg, k]], dsem).wait()
            for j in range(CHUNK // L):
                sidx = src_g[bg, k, pl.ds(j * L, L)]
                didx = dst_g[bg, k, pl.ds(j * L, L)]
                e = (plsc.load_gather(s1_v, [sidx])
                     + plsc.load_gather(s2_v, [didx]))
                e = jnp.where(e >= 0.0, e, 0.2 * e)
                nume = jnp.exp(e)
                lid = c * CHUNK + j * L + lax.iota(jnp.int32, L)
                nume = jnp.where(lid < vc, nume, 0.0)
                num_v[cb, pl.ds(j * L, L)] = nume
            pltpu.async_copy(num_v.at[cb], den_sh.at[dst_g.at[bg, k]], dsem,
                             add=True)
            if k >= 1:
                pltpu.make_async_copy(rows_v.at[1 - cb],
                                      out_sh.at[dst_g.at[bg, k]], ssem).wait()
            if k < GROUP - 1:
                pltpu.async_copy(tx_hbm.at[src_g.at[bg, k + 1]],
                                 rows_v.at[1 - cb], gsem)
            else:
                @pl.when(g + 1 < NG)
                def _():
                    pltpu.make_async_copy(
                        src_hbm.at[pl.ds(rowbase, GROUP)],
                        src_g.at[1 - bg], isem).wait()
                    pltpu.make_async_copy(
                        dst_hbm.at[pl.ds(rowbase, GROUP)],
                        dst_g.at[1 - bg], isem).wait()
                    pltpu.async_copy(tx_hbm.at[src_g.at[1 - bg, 0]],
                                     rows_v.at[1 - cb], gsem)
            pltpu.make_async_copy(tx_hbm.at[src_g.at[bg, k]],
                                  rows_v.at[cb], gsem).wait()

            def _scale(gg, inner):
                nv = num_v[cb, pl.ds(gg * L, L)]
                for l in range(L):
                    bc = jnp.full((L,), nv[l], jnp.float32)
                    rr = gg * L + l
                    for j in range(D // L):
                        sl = pl.ds(j * L, L)
                        rows_v[cb, rr, sl] = rows_v[cb, rr, sl] * bc
                return inner

            lax.fori_loop(0, CHUNK // L, _scale, 0)
            pltpu.async_copy(rows_v.at[cb], out_sh.at[dst_g.at[bg, k]], ssem,
                             add=True)
        pltpu.make_async_copy(num_v.at[0], den_sh.at[dst_g.at[bg, 0]],
                              dsem).wait()
        pltpu.make_async_copy(num_v.at[1], den_sh.at[dst_g.at[bg, 1]],
                              dsem).wait()
        pltpu.make_async_copy(rows_v.at[1], out_sh.at[dst_g.at[bg, 0]],
                              ssem).wait()

        @pl.when(g + 2 < NG)
        def _():
            rw = rowbase + (g + 2) * GROUP
            pltpu.async_copy(src_hbm.at[pl.ds(rw, GROUP)], src_g.at[bg], isem)
            pltpu.async_copy(dst_hbm.at[pl.ds(rw, GROUP)], dst_g.at[bg], isem)
        return carry

    lax.fori_loop(0, NG, _group, 0)
    plsc.subcore_barrier()

    @pl.when(sid < NS - 1)
    def _():
        pltpu.sync_copy(out_sh.at[pl.ds(base, ROW_PT)],
                        outp_hbm.at[cid, pl.ds(base, ROW_PT)])

    @pl.when(sid == NS - 1)
    def _():
        pltpu.sync_copy(out_sh.at[pl.ds(base, ROW_LAST)],
                        outp_hbm.at[cid, pl.ds(base, ROW_LAST)])

    pltpu.sync_copy(den_sh.at[pl.ds(sid * DEN_PT, DEN_PT)],
                    denp_hbm.at[cid, 0, pl.ds(sid * DEN_PT, DEN_PT)])


def _norm_body(op_ref, dp_ref, s12t_ref, tx_ref, o_ref):
    s_self = s12t_ref[:, 0:1] + s12t_ref[:, 1:2]
    s_self = jnp.where(s_self >= 0.0, s_self, 0.2 * s_self)
    selfnum = jnp.exp(s_self)
    o = op_ref[0] + op_ref[1] + selfnum * tx_ref[...]
    d = dp_ref[0] + dp_ref[1] + selfnum
    d = jnp.where(d == 0.0, 1.0, d)
    o_ref[...] = o / d


_norm = pl.pallas_call(
    _norm_body,
    out_shape=jax.ShapeDtypeStruct((N_NODES, D), jnp.float32),
)


def kernel(x, edge_index, W, a):
    ei = edge_index.astype(jnp.int32)
    src = ei[0]
    dst = ei[1]

    lid = jnp.arange(WIN * CHUNK, dtype=jnp.int32)
    vcnt = jnp.clip(E_TOT - jnp.arange(NW, dtype=jnp.int32) * EPT, 0, EPT)
    valid = lid[None, :] < vcnt[:, None]
    fill = jnp.broadcast_to(lid % N_NODES, (NW, WIN * CHUNK))

    def _tile_layout(v):
        v = jnp.pad(v, (0, EPAD - E_TOT)).reshape(NW, CPT, CHUNK)
        v = jnp.pad(v, ((0, 0), (0, WIN - CPT), (0, 0)))
        v = jnp.where(valid, v.reshape(NW, WIN * CHUNK), fill)
        return v.reshape(NW * WIN, CHUNK)

    src = _tile_layout(src)
    dst = _tile_layout(dst)
    a2 = a.reshape(2, D)

    tx, s12 = _prep(x, W, a2)
    outp, denp = _sc_edges(tx, s12[0], s12[1], src, dst)
    dp = denp[:, 0, :N_NODES, None]
    return _norm(outp, dp, s12.T, tx)


# speedup vs baseline: 1.0427x; 1.0427x over previous
import functools

import jax
import jax.numpy as jnp
from jax import lax
from jax.experimental import pallas as pl
from jax.experimental.pallas import tpu as pltpu
from jax.experimental.pallas import tpu_sc as plsc

N_NODES = 10000
D = 128
N_EDGES = 320000
E_TOT = N_EDGES + N_NODES
NC = 2
NS = 16
NW = NC * NS
L = 16
CHUNK = 96
CPT = -(-E_TOT // (NW * CHUNK))
WIN = CPT + (-CPT) % 8
GROUP = 8
NG = WIN // GROUP
EPT = CPT * CHUNK
EPAD = NW * EPT
DENP = 10240
DEN_PT = DENP // NS
ROW_PT = 624
ROW_LAST = N_NODES - 15 * ROW_PT


def _prep_body(x_ref, w_ref, a2_ref, tx_ref, s12_ref):
    tx = lax.dot_general(x_ref[...], w_ref[...], (((1,), (1,)), ((), ())),
                         preferred_element_type=jnp.float32)
    tx_ref[...] = tx
    s12_ref[...] = lax.dot_general(a2_ref[...], tx, (((1,), (1,)), ((), ())),
                                   preferred_element_type=jnp.float32)


_prep = pl.pallas_call(
    _prep_body,
    out_shape=(jax.ShapeDtypeStruct((N_NODES, D), jnp.float32),
               jax.ShapeDtypeStruct((2, N_NODES), jnp.float32)),
)


_sc_mesh = plsc.VectorSubcoreMesh(core_axis_name="c", subcore_axis_name="s")


@functools.partial(
    pl.kernel,
    mesh=_sc_mesh,
    compiler_params=pltpu.CompilerParams(use_tc_tiling_on_sc=False,
                                         needs_layout_passes=False),
    out_type=(jax.ShapeDtypeStruct((NC, N_NODES, D), jnp.float32),
              jax.ShapeDtypeStruct((NC, 1, DENP), jnp.float32)),
    scratch_types=[
        pltpu.VMEM((2, GROUP, CHUNK), jnp.int32),
        pltpu.VMEM((2, GROUP, CHUNK), jnp.int32),
        pltpu.VMEM((2, CHUNK), jnp.float32),
        pltpu.VMEM((N_NODES,), jnp.float32),
        pltpu.VMEM((N_NODES,), jnp.float32),
        pltpu.VMEM((2, CHUNK, D), jnp.float32),
        pltpu.VMEM((DEN_PT,), jnp.float32),
        pltpu.VMEM_SHARED((N_NODES, D), jnp.float32),
        pltpu.VMEM_SHARED((DENP,), jnp.float32),
        pltpu.SemaphoreType.DMA,
        pltpu.SemaphoreType.DMA,
        pltpu.SemaphoreType.DMA,
        pltpu.SemaphoreType.DMA,
    ],
)
def _sc_edges(tx_hbm, s1_hbm, s2_hbm, src_hbm, dst_hbm, outp_hbm, denp_hbm,
              src_g, dst_g, num_v, s1_v, s2_v, rows_v, zbuf, out_sh, den_sh,
              gsem, isem, dsem, ssem):
    cid = lax.axis_index("c")
    sid = lax.axis_index("s")
    w = sid * NC + cid
    zv = jnp.zeros((L,), jnp.float32)

    def _zero_rows(r, carry):
        for j in range(D // L):
            rows_v[0, r, pl.ds(j * L, L)] = zv
        return carry

    lax.fori_loop(0, CHUNK, _zero_rows, 0)

    def _zero_zbuf(i, carry):
        zbuf[pl.ds(i * L, L)] = zv
        return carry

    lax.fori_loop(0, DEN_PT // L, _zero_zbuf, 0)

    base = sid * ROW_PT

    @pl.when(sid < NS - 1)
    def _():
        off = 0
        for nrows in (CHUNK, CHUNK, CHUNK, CHUNK, CHUNK, CHUNK,
                      ROW_PT - 6 * CHUNK):
            pltpu.sync_copy(rows_v.at[0, pl.ds(0, nrows)],
                            out_sh.at[pl.ds(base + off, nrows)])
            off += nrows

    @pl.when(sid == NS - 1)
    def _():
        off = 0
        for nrows in (CHUNK, CHUNK, CHUNK, CHUNK, CHUNK, CHUNK,
                      ROW_LAST - 6 * CHUNK):
            pltpu.sync_copy(rows_v.at[0, pl.ds(0, nrows)],
                            out_sh.at[pl.ds(base + off, nrows)])
            off += nrows

    pltpu.sync_copy(zbuf, den_sh.at[pl.ds(sid * DEN_PT, DEN_PT)])
    plsc.subcore_barrier()

    rowbase = w * WIN
    vc = jnp.maximum(jnp.minimum(E_TOT - w * EPT, EPT), 0)

    pltpu.sync_copy(src_hbm.at[pl.ds(rowbase, GROUP)], src_g.at[0])
    pltpu.sync_copy(dst_hbm.at[pl.ds(rowbase, GROUP)], dst_g.at[0])
    pltpu.async_copy(tx_hbm.at[src_g.at[0, 0]], rows_v.at[0], gsem)
    pltpu.sync_copy(s1_hbm, s1_v)
    pltpu.sync_copy(s2_hbm, s2_v)
    pltpu.async_copy(src_hbm.at[pl.ds(rowbase + GROUP, GROUP)], src_g.at[1],
                     isem)
    pltpu.async_copy(dst_hbm.at[pl.ds(rowbase + GROUP, GROUP)], dst_g.at[1],
                     isem)

    def _group(g, carry):
        bg = g & 1
        for k in range(GROUP):
            c = g * GROUP + k
            cb = k & 1
            if k >= 2:
                pltpu.make_async_copy(num_v.at[cb],
                                      den_sh.at[dst_g.at[bg, k]], dsem).wait()
            for j in range(CHUNK // L):
                sidx = src_g[bg, k, pl.ds(j * L, L)]
                didx = dst_g[bg, k, pl.ds(j * L, L)]
                e = (plsc.load_gather(s1_v, [sidx])
                     + plsc.load_gather(s2_v, [didx]))
                e = jnp.where(e >= 0.0, e, 0.2 * e)
                nume = jnp.exp(e)
                lid = c * CHUNK + j * L + lax.iota(jnp.int32, L)
                nume = jnp.where(lid < vc, nume, 0.0)
                num_v[cb, pl.ds(j * L, L)] = nume
            pltpu.async_copy(num_v.at[cb], den_sh.at[dst_g.at[bg, k]], dsem,
                             add=True)
            if k >= 1:
                pltpu.make_async_copy(rows_v.at[1 - cb],
                                      out_sh.at[dst_g.at[bg, k]], ssem).wait()
            if k < GROUP - 1:
                pltpu.async_copy(tx_hbm.at[src_g.at[bg, k + 1]],
                                 rows_v.at[1 - cb], gsem)
            else:
                @pl.when(g + 1 < NG)
                def _():
                    pltpu.make_async_copy(
                        src_hbm.at[pl.ds(rowbase, GROUP)],
                        src_g.at[1 - bg], isem).wait()
                    pltpu.make_async_copy(
                        dst_hbm.at[pl.ds(rowbase, GROUP)],
                        dst_g.at[1 - bg], isem).wait()
                    pltpu.async_copy(tx_hbm.at[src_g.at[1 - bg, 0]],
                                     rows_v.at[1 - cb], gsem)
            pltpu.make_async_copy(tx_hbm.at[src_g.at[bg, k]],
                                  rows_v.at[cb], gsem).wait()

            def _scale(gg, inner):
                nv = num_v[cb, pl.ds(gg * L, L)]
                for l in range(L):
                    bc = jnp.full((L,), nv[l], jnp.float32)
                    rr = gg * L + l
                    for j in range(D // L):
                        sl = pl.ds(j * L, L)
                        rows_v[cb, rr, sl] = rows_v[cb, rr, sl] * bc
                return inner

            lax.fori_loop(0, CHUNK // L, _scale, 0)
            pltpu.async_copy(rows_v.at[cb], out_sh.at[dst_g.at[bg, k]], ssem,
                             add=True)
        pltpu.make_async_copy(num_v.at[0], den_sh.at[dst_g.at[bg, 0]],
                              dsem).wait()
        pltpu.make_async_copy(num_v.at[1], den_sh.at[dst_g.at[bg, 1]],
                              dsem).wait()
        pltpu.make_async_copy(rows_v.at[1], out_sh.at[dst_g.at[bg, 0]],
                              ssem).wait()

        @pl.when(g + 2 < NG)
        def _():
            rw = rowbase + (g + 2) * GROUP
            pltpu.async_copy(src_hbm.at[pl.ds(rw, GROUP)], src_g.at[bg], isem)
            pltpu.async_copy(dst_hbm.at[pl.ds(rw, GROUP)], dst_g.at[bg], isem)
        return carry

    lax.fori_loop(0, NG, _group, 0)
    plsc.subcore_barrier()

    @pl.when(sid < NS - 1)
    def _():
        pltpu.sync_copy(out_sh.at[pl.ds(base, ROW_PT)],
                        outp_hbm.at[cid, pl.ds(base, ROW_PT)])

    @pl.when(sid == NS - 1)
    def _():
        pltpu.sync_copy(out_sh.at[pl.ds(base, ROW_LAST)],
                        outp_hbm.at[cid, pl.ds(base, ROW_LAST)])

    pltpu.sync_copy(den_sh.at[pl.ds(sid * DEN_PT, DEN_PT)],
                    denp_hbm.at[cid, 0, pl.ds(sid * DEN_PT, DEN_PT)])


def _norm_body(op_ref, dp_ref, o_ref):
    o = op_ref[0] + op_ref[1]
    d = dp_ref[0] + dp_ref[1]
    d = jnp.where(d == 0.0, 1.0, d)
    o_ref[...] = o / d


_norm = pl.pallas_call(
    _norm_body,
    out_shape=jax.ShapeDtypeStruct((N_NODES, D), jnp.float32),
)


def kernel(x, edge_index, W, a):
    ei = edge_index.astype(jnp.int32)
    loop = jnp.arange(N_NODES, dtype=jnp.int32)
    src = jnp.concatenate([ei[0], loop])
    dst = jnp.concatenate([ei[1], loop])

    lid = jnp.arange(WIN * CHUNK, dtype=jnp.int32)
    vcnt = jnp.clip(E_TOT - jnp.arange(NW, dtype=jnp.int32) * EPT, 0, EPT)
    valid = lid[None, :] < vcnt[:, None]
    fill = jnp.broadcast_to(lid % N_NODES, (NW, WIN * CHUNK))

    def _tile_layout(v):
        v = jnp.pad(v, (0, EPAD - E_TOT)).reshape(NW, CPT, CHUNK)
        v = jnp.pad(v, ((0, 0), (0, WIN - CPT), (0, 0)))
        v = jnp.where(valid, v.reshape(NW, WIN * CHUNK), fill)
        return v.reshape(NW * WIN, CHUNK)

    src = _tile_layout(src)
    dst = _tile_layout(dst)
    a2 = a.reshape(2, D)

    tx, s12 = _prep(x, W, a2)
    outp, denp = _sc_edges(tx, s12[0], s12[1], src, dst)
    dp = denp[:, 0, :N_NODES, None]
    return _norm(outp, dp)
